# baseline (speedup 1.0000x reference)
import functools

import jax
import jax.numpy as jnp
from jax import lax
from jax.experimental import pallas as pl
from jax.experimental.pallas import tpu as pltpu
from jax.experimental.pallas import tpu_sc as plsc

B = 4096
L = 200
EMB = 64
VOCAB = 1000000
NC = 2
NS = 16
NW = NC * NS
RPW = B // NW
GROUPS = RPW // 16
C0, C1 = 104, 96
LP = 208

ROWS_SC = 640000
TBLK = 32768
NBLK = -(-VOCAB // TBLK)
MV_RPW = ROWS_SC // NW
CH = 400
MV_CHUNKS = MV_RPW // CH

_SC_PARAMS = pltpu.CompilerParams(
    needs_layout_passes=False, use_tc_tiling_on_sc=False)
_mesh = plsc.VectorSubcoreMesh(core_axis_name="c", subcore_axis_name="s")


def _matvec_body(w_ref, t_ref, o_ref):
    o_ref[...] = jax.lax.dot_general(
        w_ref[...], t_ref[...], (((1,), (0,)), ((), ())),
        preferred_element_type=jnp.float32)[None]


def _matvec_tc(W, tableT):
    out = pl.pallas_call(
        _matvec_body,
        grid=(NBLK,),
        in_specs=[
            pl.BlockSpec((1, EMB), lambda i: (0, 0)),
            pl.BlockSpec((EMB, TBLK), lambda i: (0, i)),
        ],
        out_specs=pl.BlockSpec((1, 1, TBLK), lambda i: (i, 0, 0)),
        out_shape=jax.ShapeDtypeStruct((NBLK, 1, TBLK), jnp.float32),
    )(W, tableT)
    return out.reshape(-1)


@functools.partial(
    pl.kernel,
    out_type=jax.ShapeDtypeStruct((ROWS_SC,), jnp.float32),
    mesh=_mesh,
    compiler_params=_SC_PARAMS,
    scratch_types=[
        pltpu.VMEM((2, CH, EMB), jnp.float32),
        pltpu.VMEM((EMB,), jnp.float32),
        pltpu.VMEM((MV_RPW,), jnp.float32),
        pltpu.SemaphoreType.DMA,
    ],
)
def _matvec_sc(table_hbm, w_hbm, out_hbm, buf_v, w_v, out_v, sem):
    wid = lax.axis_index("s") * NC + lax.axis_index("c")
    row0 = ROWS_TC + wid * MV_RPW

    pltpu.sync_copy(w_hbm, w_v)
    w0 = w_v[pl.ds(0, 16)]
    w1 = w_v[pl.ds(16, 16)]
    w2 = w_v[pl.ds(32, 16)]
    w3 = w_v[pl.ds(48, 16)]
    lane = lax.iota(jnp.int32, 16)
    zero = jnp.zeros((16,), jnp.float32)

    def fire(c, slot):
        pltpu.async_copy(
            table_hbm.at[pl.ds(row0 + c * CH, CH), :],
            buf_v.at[slot], sem)

    fire(0, 0)

    def chunk_body(c, _):
        @pl.when(c < MV_CHUNKS - 1)
        def _():
            fire(c + 1, jnp.bitwise_and(c + 1, 1))

        slot = jnp.bitwise_and(c, 1)
        pltpu.make_async_copy(
            table_hbm.at[pl.ds(row0, CH), :],
            buf_v.at[slot], sem).wait()

        def group_body(g, _):
            svec = zero
            for j in range(16):
                r = g * 16 + j
                p = (buf_v[slot, r, pl.ds(0, 16)] * w0
                     + buf_v[slot, r, pl.ds(16, 16)] * w1
                     + buf_v[slot, r, pl.ds(32, 16)] * w2
                     + buf_v[slot, r, pl.ds(48, 16)] * w3)
                svec = jnp.where(lane == j, jnp.sum(p), svec)
            out_v[pl.ds(c * CH + g * 16, 16)] = svec
            return 0

        lax.fori_loop(0, CH // 16, group_body, 0)
        return 0

    lax.fori_loop(0, MV_CHUNKS, chunk_body, 0)
    pltpu.sync_copy(out_v, out_hbm.at[pl.ds(wid * MV_RPW, MV_RPW)])


@functools.partial(
    pl.kernel,
    out_type=jax.ShapeDtypeStruct((B,), jnp.float32),
    mesh=_mesh,
    compiler_params=_SC_PARAMS,
    scratch_types=[
        pltpu.VMEM((RPW, L), jnp.int32),
        pltpu.VMEM((GROUPS, 16 * LP), jnp.float32),
        pltpu.VMEM((RPW,), jnp.int32),
        pltpu.VMEM((16,), jnp.float32),
        pltpu.VMEM((RPW,), jnp.float32),
        pltpu.SemaphoreType.DMA,
    ],
)
def _pool_sc(data_hbm, len_hbm, s_hbm, b_hbm, out_hbm,
             idx_v, buf_v, len_v, b_v, out_v, sem):
    wid = lax.axis_index("s") * NC + lax.axis_index("c")
    base = wid * RPW

    pltpu.sync_copy(data_hbm.at[pl.ds(base, RPW), :], idx_v)
    pltpu.sync_copy(len_hbm.at[pl.ds(base, RPW)], len_v)
    pltpu.sync_copy(b_hbm, b_v.at[pl.ds(0, 1)])

    zero = jnp.zeros((16,), jnp.float32)

    def zero_body(slot, _):
        for j in range(16):
            buf_v[slot, pl.ds(j * LP + 192, 16)] = zero
        return 0

    lax.fori_loop(0, GROUPS, zero_body, 0)

    bias = b_v[pl.ds(0, 16)][0]
    lane = lax.iota(jnp.int32, 16)

    def fire_body(g, _):
        for j in range(16):
            i = g * 16 + j
            pltpu.async_copy(
                s_hbm.at[idx_v.at[i, pl.ds(0, C0)]],
                buf_v.at[g, pl.ds(j * LP, C0)], sem)
            pltpu.async_copy(
                s_hbm.at[idx_v.at[i, pl.ds(C0, C1)]],
                buf_v.at[g, pl.ds(j * LP + C0, C1)], sem)
        return 0

    lax.fori_loop(0, GROUPS, fire_body, 0)

    def group_body(g, _):
        pltpu.make_async_copy(
            s_hbm.at[pl.ds(0, 16 * L)],
            buf_v.at[g, pl.ds(0, 16 * L)], sem).wait()

        zvec = zero
        for j in range(16):
            p = zero
            for k in range(13):
                p = p + buf_v[g, pl.ds(j * LP + k * 16, 16)]
            zvec = jnp.where(lane == j, jnp.sum(p), zvec)
        lvec = len_v[pl.ds(g * 16, 16)].astype(jnp.float32)
        zvec = zvec / lvec + bias
        out_v[pl.ds(g * 16, 16)] = 1.0 / (1.0 + jnp.exp(-zvec))
        return 0

    lax.fori_loop(0, GROUPS, group_body, 0)
    pltpu.sync_copy(out_v, out_hbm.at[pl.ds(base, RPW)])


def kernel(data, length, emb_table, W, b):
    s = _matvec_tc(W, emb_table.T)
    return _pool_sc(data, length, s, b)


# speedup vs baseline: 1.0027x; 1.0027x over previous
import functools

import jax
import jax.numpy as jnp
from jax import lax
from jax.experimental import pallas as pl
from jax.experimental.pallas import tpu as pltpu
from jax.experimental.pallas import tpu_sc as plsc

B = 4096
L = 200
EMB = 64
VOCAB = 1000000
NC = 2
NS = 16
NW = NC * NS
RPW = B // NW
GROUPS = RPW // 16
C0, C1 = 104, 96
LP = 208

TBLK = 32768
NBLK = -(-VOCAB // TBLK)
_SC_PARAMS = pltpu.CompilerParams(
    needs_layout_passes=False, use_tc_tiling_on_sc=False)
_mesh = plsc.VectorSubcoreMesh(core_axis_name="c", subcore_axis_name="s")


def _matvec_body(w_ref, t_ref, o_ref):
    o_ref[...] = jax.lax.dot_general(
        w_ref[...], t_ref[...], (((1,), (0,)), ((), ())),
        preferred_element_type=jnp.float32)[None]


def _matvec_tc(W, tableT):
    out = pl.pallas_call(
        _matvec_body,
        grid=(NBLK,),
        in_specs=[
            pl.BlockSpec((1, EMB), lambda i: (0, 0)),
            pl.BlockSpec((EMB, TBLK), lambda i: (0, i)),
        ],
        out_specs=pl.BlockSpec((1, 1, TBLK), lambda i: (i, 0, 0)),
        out_shape=jax.ShapeDtypeStruct((NBLK, 1, TBLK), jnp.float32),
    )(W, tableT)
    return out.reshape(-1)


@functools.partial(
    pl.kernel,
    out_type=jax.ShapeDtypeStruct((B,), jnp.float32),
    mesh=_mesh,
    compiler_params=_SC_PARAMS,
    scratch_types=[
        pltpu.VMEM((RPW * L,), jnp.int32),
        pltpu.VMEM((GROUPS, 16 * LP), jnp.float32),
        pltpu.VMEM((RPW,), jnp.int32),
        pltpu.VMEM((16,), jnp.float32),
        pltpu.VMEM((RPW,), jnp.float32),
        pltpu.SemaphoreType.DMA,
    ],
)
def _pool_sc(data_hbm, len_hbm, s_hbm, b_hbm, out_hbm,
             idx_v, buf_v, len_v, b_v, out_v, sem):
    wid = lax.axis_index("s") * NC + lax.axis_index("c")
    base = wid * RPW

    pltpu.sync_copy(data_hbm.at[pl.ds(base * L, RPW * L)], idx_v)
    pltpu.sync_copy(len_hbm.at[pl.ds(base, RPW)], len_v)
    pltpu.sync_copy(b_hbm, b_v.at[pl.ds(0, 1)])

    zero = jnp.zeros((16,), jnp.float32)

    def zero_body(slot, _):
        for j in range(16):
            buf_v[slot, pl.ds(j * LP + 192, 16)] = zero
        return 0

    lax.fori_loop(0, GROUPS, zero_body, 0)

    bias = b_v[pl.ds(0, 16)][0]
    lane = lax.iota(jnp.int32, 16)

    def fire_body(g, _):
        for j in range(16):
            i = g * 16 + j
            pltpu.async_copy(
                s_hbm.at[idx_v.at[pl.ds(i * L, C0)]],
                buf_v.at[g, pl.ds(j * LP, C0)], sem)
            pltpu.async_copy(
                s_hbm.at[idx_v.at[pl.ds(i * L + C0, C1)]],
                buf_v.at[g, pl.ds(j * LP + C0, C1)], sem)
        return 0

    lax.fori_loop(0, GROUPS, fire_body, 0)

    def group_body(g, _):
        pltpu.make_async_copy(
            s_hbm.at[pl.ds(0, 16 * L)],
            buf_v.at[g, pl.ds(0, 16 * L)], sem).wait()

        zvec = zero
        for j in range(16):
            p = zero
            for k in range(13):
                p = p + buf_v[g, pl.ds(j * LP + k * 16, 16)]
            zvec = jnp.where(lane == j, jnp.sum(p), zvec)
        lvec = len_v[pl.ds(g * 16, 16)].astype(jnp.float32)
        zvec = zvec / lvec + bias
        out_v[pl.ds(g * 16, 16)] = 1.0 / (1.0 + jnp.exp(-zvec))
        return 0

    lax.fori_loop(0, GROUPS, group_body, 0)
    pltpu.sync_copy(out_v, out_hbm.at[pl.ds(base, RPW)])


def kernel(data, length, emb_table, W, b):
    s = _matvec_tc(W, emb_table.T)
    return _pool_sc(data.reshape(-1), length, s, b)
